# baseline (speedup 1.0000x reference)
import functools

import jax
import jax.numpy as jnp
from jax import lax
from jax.experimental import pallas as pl
from jax.experimental.pallas import tpu as pltpu
from jax.experimental.pallas import tpu_sc as plsc

_NUM_CLASSES = 1024
_FEAT = 256
_BATCH = 16384
_ALPHA = 0.5
_THETA = 10.0

_B_TC = 8192
_B_SC = _BATCH - _B_TC

_L = 16
_NC = 2
_NS = 16
_NW = _NC * _NS
_BPW = _B_SC // _NW
_CK = 64
_NCH = _BPW // _CK
_FV = _FEAT // _L

_TC_BLK = 1024


def _sc_sumsq(x, labels, centers):
    mesh = plsc.VectorSubcoreMesh(core_axis_name="c", subcore_axis_name="s")

    @functools.partial(
        pl.kernel,
        out_type=jax.ShapeDtypeStruct((_NW * _L,), jnp.float32),
        mesh=mesh,
        scratch_types=[
            pltpu.VMEM((_BPW,), jnp.int32),
            pltpu.VMEM((_CK, _FEAT), jnp.float32),
            pltpu.VMEM((_CK, _FEAT), jnp.float32),
            pltpu.VMEM((_CK, _FEAT), jnp.float32),
            pltpu.VMEM((_CK, _FEAT), jnp.float32),
            pltpu.VMEM((_L,), jnp.float32),
            pltpu.SemaphoreType.DMA,
            pltpu.SemaphoreType.DMA,
            pltpu.SemaphoreType.DMA,
            pltpu.SemaphoreType.DMA,
        ],
    )
    def body(x_hbm, lbl_hbm, cen_hbm, out_hbm,
             lbl_v, xb0, xb1, cb0, cb1, ob, sx0, sx1, sc0, sc1):
        wid = lax.axis_index("s") * _NC + lax.axis_index("c")
        base = wid * _BPW
        pltpu.sync_copy(lbl_hbm.at[pl.ds(_B_TC + base, _BPW)], lbl_v)

        xbufs = (xb0, xb1)
        cbufs = (cb0, cb1)
        xsems = (sx0, sx1)
        csems = (sc0, sc1)

        def start(g, slot):
            row0 = _B_TC + base + g * _CK
            pltpu.async_copy(x_hbm.at[pl.ds(row0, _CK)], xbufs[slot], xsems[slot])
            pltpu.async_copy(cen_hbm.at[lbl_v.at[pl.ds(g * _CK, _CK)]],
                             cbufs[slot], csems[slot])

        def wait(g, slot):
            row0 = _B_TC + base + g * _CK
            pltpu.make_async_copy(x_hbm.at[pl.ds(row0, _CK)],
                                  xbufs[slot], xsems[slot]).wait()
            pltpu.make_async_copy(cen_hbm.at[lbl_v.at[pl.ds(g * _CK, _CK)]],
                                  cbufs[slot], csems[slot]).wait()

        lanes = lax.iota(jnp.int32, _L)
        perms = [lanes ^ sh for sh in (8, 4, 2, 1)]
        nacc = 4

        def rsqrt_nr(v):
            yi = jnp.int32(0x5F3759DF) - lax.shift_right_logical(
                lax.bitcast_convert_type(v, jnp.int32), 1)
            y = lax.bitcast_convert_type(yi, jnp.float32)
            for _ in range(3):
                y = y * (1.5 - 0.5 * v * y * y)
            return y

        total = jnp.zeros((_L,), jnp.float32)
        start(0, 0)
        for g in range(_NCH):
            slot = g % 2
            if g + 1 < _NCH:
                start(g + 1, (g + 1) % 2)
            wait(g, slot)
            xb, cb = xbufs[slot], cbufs[slot]

            def grp_body(gi, tot, xb=xb, cb=cb):
                def row_body(k, vec, gi=gi, xb=xb, cb=cb):
                    r = gi * _L + k
                    accs = [jnp.zeros((_L,), jnp.float32)
                            for _ in range(nacc)]
                    for f in range(_FV):
                        xv = xb[r, pl.ds(f * _L, _L)]
                        cv = cb[r, pl.ds(f * _L, _L)]
                        d = xv - cv
                        accs[f % nacc] = accs[f % nacc] + d * d
                    acc = (accs[0] + accs[1]) + (accs[2] + accs[3])
                    for perm in perms:
                        acc = acc + acc.at[perm].get(mode="promise_in_bounds")
                    m = lanes == lax.broadcast_in_dim(k, (_L,), ())
                    return jnp.where(m, acc, vec)

                vec = lax.fori_loop(0, _L, row_body,
                                    jnp.zeros((_L,), jnp.float32), unroll=2)
                v = vec + 1e-30
                return tot + v * rsqrt_nr(v)

            total = lax.fori_loop(0, _CK // _L, grp_body, total)

        ob[...] = total
        pltpu.sync_copy(ob, out_hbm.at[pl.ds(wid * _L, _L)])

    return body(x, labels, centers)


def _rank_loss(cd_ref):
    nc = _NUM_CLASSES
    col1 = cd_ref[:, 1:2]
    col2 = cd_ref[:, 2:3]
    d03 = cd_ref[0:1, 3:4]
    rows = lax.broadcasted_iota(jnp.int32, (nc - 1, 1), 0)
    pad_hi = jnp.full((1, 1), 1e30, jnp.float32)
    a_full = jnp.concatenate(
        [jnp.where(rows < nc - 2, col2, 1e30), pad_hi], axis=0)
    b_col = jnp.concatenate([col1, -pad_hi], axis=0)
    bmat = lax.dot_general(
        jnp.ones((32, 1), jnp.float32), b_col,
        (((1,), (1,)), ((), ())),
        preferred_element_type=jnp.float32)
    acc = jnp.zeros((32, nc), jnp.float32)
    for i in range(nc // 32):
        a = lax.slice(a_full, (i * 32, 0), ((i + 1) * 32, 1))
        acc = acc + jnp.maximum(_THETA - a + bmat, 0.0)
    rk1 = jnp.sum(acc)
    rk2 = jnp.sum(jnp.maximum(2.0 * _THETA - d03 + b_col, 0.0))
    return rk1 + rk2


def _tc_partial(x, labels3d, ctb, cd):
    nblk = _B_TC // _TC_BLK

    def body(x_ref, lbl_ref, ctb_ref, cd_ref, o_ref, rk_ref):
        i = pl.program_id(0)
        lbl = lbl_ref[...].reshape(1, _TC_BLK)
        cls = lax.broadcasted_iota(jnp.int32, (_NUM_CLASSES, _TC_BLK), 0)
        onehot_t = jnp.where(cls == lbl, 1.0, 0.0).astype(jnp.bfloat16)
        c_nat = lax.dot_general(
            onehot_t, ctb_ref[...],
            (((0,), (0,)), ((), ())),
            preferred_element_type=jnp.float32)
        d = x_ref[...] - c_nat
        dist = jnp.sqrt(jnp.sum(d * d, axis=1, keepdims=True))
        part = jnp.sum(dist).reshape(1, 1)

        @pl.when(i == 0)
        def _():
            o_ref[...] = part
            rk_ref[...] = _rank_loss(cd_ref).reshape(1, 1)

        @pl.when(i > 0)
        def _():
            o_ref[...] = o_ref[...] + part

    return pl.pallas_call(
        body,
        grid=(nblk,),
        in_specs=[
            pl.BlockSpec((_TC_BLK, _FEAT), lambda i: (i, 0)),
            pl.BlockSpec((1, 1, _TC_BLK), lambda i: (i, 0, 0)),
            pl.BlockSpec((_NUM_CLASSES, _FEAT), lambda i: (0, 0)),
            pl.BlockSpec((_NUM_CLASSES - 1, 128), lambda i: (0, 0)),
        ],
        out_specs=[
            pl.BlockSpec((1, 1), lambda i: (0, 0)),
            pl.BlockSpec((1, 1), lambda i: (0, 0)),
        ],
        out_shape=[
            jax.ShapeDtypeStruct((1, 1), jnp.float32),
            jax.ShapeDtypeStruct((1, 1), jnp.float32),
        ],
    )(x, labels3d, ctb, cd)


def _tc_finalize(scp, tc_part, rk):
    def body(sc_ref, tp_ref, rk_ref, o_ref):
        dist_sum = jnp.sum(sc_ref[...]) + tp_ref[0, 0]
        loss = 0.5 * dist_sum / _BATCH
        o_ref[...] = (_ALPHA * (loss + rk_ref[0, 0])).reshape(1, 1)

    return pl.pallas_call(
        body,
        grid=(1,),
        in_specs=[
            pl.BlockSpec((_NW * _L // 128, 128), lambda i: (0, 0)),
            pl.BlockSpec((1, 1), lambda i: (0, 0)),
            pl.BlockSpec((1, 1), lambda i: (0, 0)),
        ],
        out_specs=pl.BlockSpec((1, 1), lambda i: (0, 0)),
        out_shape=jax.ShapeDtypeStruct((1, 1), jnp.float32),
    )(scp, tc_part, rk)


def kernel(x, labels, centers, centers_distances):
    sc_partials = _sc_sumsq(x, labels, centers)
    scp = sc_partials.reshape(_NW * _L // 128, 128)
    ctb = centers.astype(jnp.bfloat16)
    labels3d = labels[:_B_TC].reshape(_B_TC // _TC_BLK, 1, _TC_BLK)
    tc_part, rk = _tc_partial(x, labels3d, ctb, centers_distances)
    out = _tc_finalize(scp, tc_part, rk)
    return out[0, 0]


# speedup vs baseline: 1.0191x; 1.0191x over previous
import functools

import jax
import jax.numpy as jnp
from jax import lax
from jax.experimental import pallas as pl
from jax.experimental.pallas import tpu as pltpu
from jax.experimental.pallas import tpu_sc as plsc

_NUM_CLASSES = 1024
_FEAT = 256
_BATCH = 16384
_ALPHA = 0.5
_THETA = 10.0

_B_TC = 8192
_B_SC = _BATCH - _B_TC

_L = 16
_NC = 2
_NS = 16
_NW = _NC * _NS
_BPW = _B_SC // _NW
_CK = 64
_NCH = _BPW // _CK
_FV = _FEAT // _L

_TC_BLK = 1024


def _sc_sumsq(x, labels, centers):
    mesh = plsc.VectorSubcoreMesh(core_axis_name="c", subcore_axis_name="s")

    @functools.partial(
        pl.kernel,
        out_type=jax.ShapeDtypeStruct((_NW * _L,), jnp.float32),
        mesh=mesh,
        scratch_types=[
            pltpu.VMEM((_BPW,), jnp.int32),
            pltpu.VMEM((_CK, _FEAT), jnp.float32),
            pltpu.VMEM((_CK, _FEAT), jnp.float32),
            pltpu.VMEM((_CK, _FEAT), jnp.float32),
            pltpu.VMEM((_CK, _FEAT), jnp.float32),
            pltpu.VMEM((_L,), jnp.float32),
            pltpu.SemaphoreType.DMA,
            pltpu.SemaphoreType.DMA,
            pltpu.SemaphoreType.DMA,
            pltpu.SemaphoreType.DMA,
        ],
    )
    def body(x_hbm, lbl_hbm, cen_hbm, out_hbm,
             lbl_v, xb0, xb1, cb0, cb1, ob, sx0, sx1, sc0, sc1):
        wid = lax.axis_index("s") * _NC + lax.axis_index("c")
        base = wid * _BPW
        pltpu.sync_copy(lbl_hbm.at[pl.ds(_B_TC + base, _BPW)], lbl_v)

        xbufs = (xb0, xb1)
        cbufs = (cb0, cb1)
        xsems = (sx0, sx1)
        csems = (sc0, sc1)

        def start(g, slot):
            row0 = _B_TC + base + g * _CK
            pltpu.async_copy(x_hbm.at[pl.ds(row0, _CK)], xbufs[slot], xsems[slot])
            pltpu.async_copy(cen_hbm.at[lbl_v.at[pl.ds(g * _CK, _CK)]],
                             cbufs[slot], csems[slot])

        def wait(g, slot):
            row0 = _B_TC + base + g * _CK
            pltpu.make_async_copy(x_hbm.at[pl.ds(row0, _CK)],
                                  xbufs[slot], xsems[slot]).wait()
            pltpu.make_async_copy(cen_hbm.at[lbl_v.at[pl.ds(g * _CK, _CK)]],
                                  cbufs[slot], csems[slot]).wait()

        lanes = lax.iota(jnp.int32, _L)
        perms = [lanes ^ sh for sh in (8, 4, 2, 1)]
        nacc = 4

        def rsqrt_nr(v):
            yi = jnp.int32(0x5F3759DF) - lax.shift_right_logical(
                lax.bitcast_convert_type(v, jnp.int32), 1)
            y = lax.bitcast_convert_type(yi, jnp.float32)
            for _ in range(3):
                y = y * (1.5 - 0.5 * v * y * y)
            return y

        total = jnp.zeros((_L,), jnp.float32)
        start(0, 0)
        for g in range(_NCH):
            slot = g % 2
            if g + 1 < _NCH:
                start(g + 1, (g + 1) % 2)
            wait(g, slot)
            xb, cb = xbufs[slot], cbufs[slot]

            def grp_body(gi, tot, xb=xb, cb=cb):
                def row_body(k, vec, gi=gi, xb=xb, cb=cb):
                    r = gi * _L + k
                    accs = [jnp.zeros((_L,), jnp.float32)
                            for _ in range(nacc)]
                    for f in range(_FV):
                        xv = xb[r, pl.ds(f * _L, _L)]
                        cv = cb[r, pl.ds(f * _L, _L)]
                        d = xv - cv
                        accs[f % nacc] = accs[f % nacc] + d * d
                    acc = (accs[0] + accs[1]) + (accs[2] + accs[3])
                    for perm in perms:
                        acc = acc + acc.at[perm].get(mode="promise_in_bounds")
                    m = lanes == lax.broadcast_in_dim(k, (_L,), ())
                    return jnp.where(m, acc, vec)

                vec = lax.fori_loop(0, _L, row_body,
                                    jnp.zeros((_L,), jnp.float32))
                v = vec + 1e-30
                return tot + v * rsqrt_nr(v)

            total = lax.fori_loop(0, _CK // _L, grp_body, total)

        ob[...] = total
        pltpu.sync_copy(ob, out_hbm.at[pl.ds(wid * _L, _L)])

    return body(x, labels, centers)


def _rank_loss(cd_ref):
    nc = _NUM_CLASSES
    col1 = cd_ref[:, 1:2]
    col2 = cd_ref[:, 2:3]
    d03 = cd_ref[0:1, 3:4]
    rows = lax.broadcasted_iota(jnp.int32, (nc - 1, 1), 0)
    pad_hi = jnp.full((1, 1), 1e30, jnp.float32)
    a_full = jnp.concatenate(
        [jnp.where(rows < nc - 2, col2, 1e30), pad_hi], axis=0)
    b_col = jnp.concatenate([col1, -pad_hi], axis=0)
    bmat = lax.dot_general(
        jnp.ones((32, 1), jnp.float32), b_col,
        (((1,), (1,)), ((), ())),
        preferred_element_type=jnp.float32)
    acc = jnp.zeros((32, nc), jnp.float32)
    for i in range(nc // 32):
        a = lax.slice(a_full, (i * 32, 0), ((i + 1) * 32, 1))
        acc = acc + jnp.maximum(_THETA - a + bmat, 0.0)
    rk1 = jnp.sum(acc)
    rk2 = jnp.sum(jnp.maximum(2.0 * _THETA - d03 + b_col, 0.0))
    return rk1 + rk2


def _tc_partial(x, labels3d, ctb, cd):
    nblk = _B_TC // _TC_BLK

    def body(x_ref, lbl_ref, ctb_ref, cd_ref, o_ref, rk_ref):
        i = pl.program_id(0)
        lbl = lbl_ref[...].reshape(1, _TC_BLK)
        cls = lax.broadcasted_iota(jnp.int32, (_NUM_CLASSES, _TC_BLK), 0)
        onehot_t = jnp.where(cls == lbl, 1.0, 0.0).astype(jnp.bfloat16)
        c_nat = lax.dot_general(
            onehot_t, ctb_ref[...],
            (((0,), (0,)), ((), ())),
            preferred_element_type=jnp.float32)
        d = x_ref[...] - c_nat
        dist = jnp.sqrt(jnp.sum(d * d, axis=1, keepdims=True))
        part = jnp.sum(dist).reshape(1, 1)

        @pl.when(i == 0)
        def _():
            o_ref[...] = part
            rk_ref[...] = _rank_loss(cd_ref).reshape(1, 1)

        @pl.when(i > 0)
        def _():
            o_ref[...] = o_ref[...] + part

    return pl.pallas_call(
        body,
        grid=(nblk,),
        in_specs=[
            pl.BlockSpec((_TC_BLK, _FEAT), lambda i: (i, 0)),
            pl.BlockSpec((1, 1, _TC_BLK), lambda i: (i, 0, 0)),
            pl.BlockSpec((_NUM_CLASSES, _FEAT), lambda i: (0, 0)),
            pl.BlockSpec((_NUM_CLASSES - 1, 128), lambda i: (0, 0)),
        ],
        out_specs=[
            pl.BlockSpec((1, 1), lambda i: (0, 0)),
            pl.BlockSpec((1, 1), lambda i: (0, 0)),
        ],
        out_shape=[
            jax.ShapeDtypeStruct((1, 1), jnp.float32),
            jax.ShapeDtypeStruct((1, 1), jnp.float32),
        ],
    )(x, labels3d, ctb, cd)


def _tc_finalize(scp, tc_part, rk):
    def body(sc_ref, tp_ref, rk_ref, o_ref):
        dist_sum = jnp.sum(sc_ref[...]) + tp_ref[0, 0]
        loss = 0.5 * dist_sum / _BATCH
        o_ref[...] = (_ALPHA * (loss + rk_ref[0, 0])).reshape(1, 1)

    return pl.pallas_call(
        body,
        out_shape=jax.ShapeDtypeStruct((1, 1), jnp.float32),
    )(scp, tc_part, rk)


def kernel(x, labels, centers, centers_distances):
    sc_partials = _sc_sumsq(x, labels, centers)
    scp = sc_partials.reshape(_NW * _L // 128, 128)
    ctb = centers.astype(jnp.bfloat16)
    labels3d = labels[:_B_TC].reshape(_B_TC // _TC_BLK, 1, _TC_BLK)
    tc_part, rk = _tc_partial(x, labels3d, ctb, centers_distances)
    out = _tc_finalize(scp, tc_part, rk)
    return out[0, 0]


# speedup vs baseline: 1.0252x; 1.0060x over previous
import functools

import jax
import jax.numpy as jnp
from jax import lax
from jax.experimental import pallas as pl
from jax.experimental.pallas import tpu as pltpu
from jax.experimental.pallas import tpu_sc as plsc

_NUM_CLASSES = 1024
_FEAT = 256
_BATCH = 16384
_ALPHA = 0.5
_THETA = 10.0

_B_TC = 8192
_B_SC = _BATCH - _B_TC

_L = 16
_NC = 2
_NS = 16
_NW = _NC * _NS
_BPW = _B_SC // _NW
_CK = 64
_NCH = _BPW // _CK
_FV = _FEAT // _L

_TC_BLK = 2048


def _sc_sumsq(x, labels, centers):
    mesh = plsc.VectorSubcoreMesh(core_axis_name="c", subcore_axis_name="s")

    @functools.partial(
        pl.kernel,
        out_type=jax.ShapeDtypeStruct((_NW * _L,), jnp.float32),
        mesh=mesh,
        scratch_types=[
            pltpu.VMEM((_BPW,), jnp.int32),
            pltpu.VMEM((_CK, _FEAT), jnp.float32),
            pltpu.VMEM((_CK, _FEAT), jnp.float32),
            pltpu.VMEM((_CK, _FEAT), jnp.float32),
            pltpu.VMEM((_CK, _FEAT), jnp.float32),
            pltpu.VMEM((_L,), jnp.float32),
            pltpu.SemaphoreType.DMA,
            pltpu.SemaphoreType.DMA,
            pltpu.SemaphoreType.DMA,
            pltpu.SemaphoreType.DMA,
        ],
    )
    def body(x_hbm, lbl_hbm, cen_hbm, out_hbm,
             lbl_v, xb0, xb1, cb0, cb1, ob, sx0, sx1, sc0, sc1):
        wid = lax.axis_index("s") * _NC + lax.axis_index("c")
        base = wid * _BPW
        pltpu.sync_copy(lbl_hbm.at[pl.ds(_B_TC + base, _BPW)], lbl_v)

        xbufs = (xb0, xb1)
        cbufs = (cb0, cb1)
        xsems = (sx0, sx1)
        csems = (sc0, sc1)

        def start(g, slot):
            row0 = _B_TC + base + g * _CK
            pltpu.async_copy(x_hbm.at[pl.ds(row0, _CK)], xbufs[slot], xsems[slot])
            pltpu.async_copy(cen_hbm.at[lbl_v.at[pl.ds(g * _CK, _CK)]],
                             cbufs[slot], csems[slot])

        def wait(g, slot):
            row0 = _B_TC + base + g * _CK
            pltpu.make_async_copy(x_hbm.at[pl.ds(row0, _CK)],
                                  xbufs[slot], xsems[slot]).wait()
            pltpu.make_async_copy(cen_hbm.at[lbl_v.at[pl.ds(g * _CK, _CK)]],
                                  cbufs[slot], csems[slot]).wait()

        lanes = lax.iota(jnp.int32, _L)
        perms = [lanes ^ sh for sh in (8, 4, 2, 1)]
        nacc = 4

        def rsqrt_nr(v):
            yi = jnp.int32(0x5F3759DF) - lax.shift_right_logical(
                lax.bitcast_convert_type(v, jnp.int32), 1)
            y = lax.bitcast_convert_type(yi, jnp.float32)
            for _ in range(3):
                y = y * (1.5 - 0.5 * v * y * y)
            return y

        total = jnp.zeros((_L,), jnp.float32)
        start(0, 0)
        for g in range(_NCH):
            slot = g % 2
            if g + 1 < _NCH:
                start(g + 1, (g + 1) % 2)
            wait(g, slot)
            xb, cb = xbufs[slot], cbufs[slot]

            def grp_body(gi, tot, xb=xb, cb=cb):
                def row_body(k, vec, gi=gi, xb=xb, cb=cb):
                    r = gi * _L + k
                    accs = [jnp.zeros((_L,), jnp.float32)
                            for _ in range(nacc)]
                    for f in range(_FV):
                        xv = xb[r, pl.ds(f * _L, _L)]
                        cv = cb[r, pl.ds(f * _L, _L)]
                        d = xv - cv
                        accs[f % nacc] = accs[f % nacc] + d * d
                    acc = (accs[0] + accs[1]) + (accs[2] + accs[3])
                    for perm in perms:
                        acc = acc + acc.at[perm].get(mode="promise_in_bounds")
                    m = lanes == lax.broadcast_in_dim(k, (_L,), ())
                    return jnp.where(m, acc, vec)

                vec = lax.fori_loop(0, _L, row_body,
                                    jnp.zeros((_L,), jnp.float32))
                v = vec + 1e-30
                return tot + v * rsqrt_nr(v)

            total = lax.fori_loop(0, _CK // _L, grp_body, total)

        ob[...] = total
        pltpu.sync_copy(ob, out_hbm.at[pl.ds(wid * _L, _L)])

    return body(x, labels, centers)


def _rank_loss(cd_ref):
    nc = _NUM_CLASSES
    col1 = cd_ref[:, 1:2]
    col2 = cd_ref[:, 2:3]
    d03 = cd_ref[0:1, 3:4]
    rows = lax.broadcasted_iota(jnp.int32, (nc - 1, 1), 0)
    pad_hi = jnp.full((1, 1), 1e30, jnp.float32)
    a_full = jnp.concatenate(
        [jnp.where(rows < nc - 2, col2, 1e30), pad_hi], axis=0)
    b_col = jnp.concatenate([col1, -pad_hi], axis=0)
    bmat = lax.dot_general(
        jnp.ones((32, 1), jnp.float32), b_col,
        (((1,), (1,)), ((), ())),
        preferred_element_type=jnp.float32)
    acc = jnp.zeros((32, nc), jnp.float32)
    for i in range(nc // 32):
        a = lax.slice(a_full, (i * 32, 0), ((i + 1) * 32, 1))
        acc = acc + jnp.maximum(_THETA - a + bmat, 0.0)
    rk1 = jnp.sum(acc)
    rk2 = jnp.sum(jnp.maximum(2.0 * _THETA - d03 + b_col, 0.0))
    return rk1 + rk2


def _tc_partial(x, labels3d, ctb, cd):
    nblk = _B_TC // _TC_BLK

    def body(x_ref, lbl_ref, ctb_ref, cd_ref, o_ref, rk_ref):
        i = pl.program_id(0)
        lbl = lbl_ref[...].reshape(1, _TC_BLK)
        cls = lax.broadcasted_iota(jnp.int32, (_NUM_CLASSES, _TC_BLK), 0)
        onehot_t = jnp.where(cls == lbl, 1.0, 0.0).astype(jnp.bfloat16)
        c_nat = lax.dot_general(
            onehot_t, ctb_ref[...],
            (((0,), (0,)), ((), ())),
            preferred_element_type=jnp.float32)
        d = x_ref[...] - c_nat
        dist = jnp.sqrt(jnp.sum(d * d, axis=1, keepdims=True))
        part = jnp.sum(dist).reshape(1, 1)

        @pl.when(i == 0)
        def _():
            o_ref[...] = part
            rk_ref[...] = _rank_loss(cd_ref).reshape(1, 1)

        @pl.when(i > 0)
        def _():
            o_ref[...] = o_ref[...] + part

    return pl.pallas_call(
        body,
        grid=(nblk,),
        in_specs=[
            pl.BlockSpec((_TC_BLK, _FEAT), lambda i: (i, 0)),
            pl.BlockSpec((1, 1, _TC_BLK), lambda i: (i, 0, 0)),
            pl.BlockSpec((_NUM_CLASSES, _FEAT), lambda i: (0, 0)),
            pl.BlockSpec((_NUM_CLASSES - 1, 128), lambda i: (0, 0)),
        ],
        out_specs=[
            pl.BlockSpec((1, 1), lambda i: (0, 0)),
            pl.BlockSpec((1, 1), lambda i: (0, 0)),
        ],
        out_shape=[
            jax.ShapeDtypeStruct((1, 1), jnp.float32),
            jax.ShapeDtypeStruct((1, 1), jnp.float32),
        ],
    )(x, labels3d, ctb, cd)


def _tc_finalize(scp, tc_part, rk):
    def body(sc_ref, tp_ref, rk_ref, o_ref):
        dist_sum = jnp.sum(sc_ref[...]) + tp_ref[0, 0]
        loss = 0.5 * dist_sum / _BATCH
        o_ref[...] = (_ALPHA * (loss + rk_ref[0, 0])).reshape(1, 1)

    return pl.pallas_call(
        body,
        out_shape=jax.ShapeDtypeStruct((1, 1), jnp.float32),
    )(scp, tc_part, rk)


def kernel(x, labels, centers, centers_distances):
    sc_partials = _sc_sumsq(x, labels, centers)
    scp = sc_partials.reshape(_NW * _L // 128, 128)
    ctb = centers.astype(jnp.bfloat16)
    labels3d = labels[:_B_TC].reshape(_B_TC // _TC_BLK, 1, _TC_BLK)
    tc_part, rk = _tc_partial(x, labels3d, ctb, centers_distances)
    out = _tc_finalize(scp, tc_part, rk)
    return out[0, 0]
